# baseline (speedup 1.0000x reference)
import functools

import jax
import jax.numpy as jnp
from jax import lax
from jax.experimental import pallas as pl
from jax.experimental.pallas import tpu as pltpu
from jax.experimental.pallas import tpu_sc as plsc

VOCAB = 50265
HIDDEN = 768
MAX_POS = 514
PAD_IDX = 1
EPS = 1e-5
B, S = 64, 512

NC, NS, L = 2, 16, 16
NW = NC * NS
ROWS_PER_W = B // NW
CH = 64
NCH = S // CH
JV = HIDDEN // L


def kernel(input_ids, token_type_ids, word_emb, pos_emb, type_emb, ln_gamma, ln_beta):
    mesh = plsc.VectorSubcoreMesh(
        core_axis_name="c", subcore_axis_name="s", num_cores=NC, num_subcores=NS
    )

    @functools.partial(
        pl.kernel,
        out_type=jax.ShapeDtypeStruct((B, S, HIDDEN), jnp.float32),
        mesh=mesh,
        scratch_types=[
            pltpu.VMEM((S,), jnp.int32),
            pltpu.VMEM((S,), jnp.int32),
            pltpu.VMEM((CH, HIDDEN), jnp.float32),
            pltpu.VMEM((CH, HIDDEN), jnp.float32),
            pltpu.VMEM((HIDDEN,), jnp.float32),
            pltpu.VMEM((HIDDEN,), jnp.float32),
            pltpu.VMEM((HIDDEN,), jnp.float32),
            pltpu.SemaphoreType.DMA,
            pltpu.SemaphoreType.DMA,
        ],
        compiler_params=pltpu.CompilerParams(needs_layout_passes=False),
    )
    def emb_kernel(ids_hbm, tt_hbm, wemb_hbm, pemb_hbm, temb_hbm, g_hbm, b_hbm,
                   out_hbm, ids_v, pos_v, wrows, prows, type_v, gamma_v, beta_v,
                   sem_w, sem_p):
        del tt_hbm
        wid = lax.axis_index("s") * NC + lax.axis_index("c")

        pltpu.sync_copy(temb_hbm.at[0], type_v)
        pltpu.sync_copy(g_hbm, gamma_v)
        pltpu.sync_copy(b_hbm, beta_v)

        def do_row(rr, _):
            r = wid * ROWS_PER_W + rr
            pltpu.sync_copy(ids_hbm.at[r], ids_v)

            def cs_body(j, carry):
                v = ids_v[pl.ds(j * L, L)]
                m = jnp.where(v != PAD_IDX, jnp.int32(1), jnp.int32(0))
                c = plsc.cumsum(m) + carry
                pos_v[pl.ds(j * L, L)] = c * m + 1
                return carry + jnp.sum(m)

            lax.fori_loop(0, S // L, cs_body, jnp.int32(0), unroll=2)

            def do_chunk(chk, _):
                c0 = pl.multiple_of(chk * CH, CH)
                cp_w = pltpu.async_copy(
                    wemb_hbm.at[ids_v.at[pl.ds(c0, CH)]], wrows, sem_w)
                cp_p = pltpu.async_copy(
                    pemb_hbm.at[pos_v.at[pl.ds(c0, CH)]], prows, sem_p)
                cp_w.wait()
                cp_p.wait()

                def token_body(t, _):
                    s = jnp.zeros((L,), jnp.float32)
                    s2 = jnp.zeros((L,), jnp.float32)
                    for j in range(JV):
                        x = (wrows[t, pl.ds(j * L, L)]
                             + prows[t, pl.ds(j * L, L)]
                             + type_v[pl.ds(j * L, L)])
                        wrows[t, pl.ds(j * L, L)] = x
                        s = s + x
                        s2 = s2 + x * x
                    mu_v = jnp.full((L,), jnp.sum(s), jnp.float32) * (1.0 / HIDDEN)
                    s2_v = jnp.full((L,), jnp.sum(s2), jnp.float32) * (1.0 / HIDDEN)
                    vv = s2_v - mu_v * mu_v + EPS
                    yi = jnp.int32(0x5F3759DF) - (plsc.bitcast(vv, jnp.int32) >> 1)
                    y = plsc.bitcast(yi, jnp.float32)
                    for _ in range(3):
                        y = y * (1.5 - 0.5 * vv * y * y)
                    for j in range(JV):
                        x = wrows[t, pl.ds(j * L, L)]
                        g = gamma_v[pl.ds(j * L, L)]
                        bb = beta_v[pl.ds(j * L, L)]
                        wrows[t, pl.ds(j * L, L)] = (x - mu_v) * y * g + bb
                    return 0

                lax.fori_loop(0, CH, token_body, 0)
                pltpu.sync_copy(wrows, out_hbm.at[r, pl.ds(c0, CH)])
                return 0

            lax.fori_loop(0, NCH, do_chunk, 0)
            return 0

        lax.fori_loop(0, ROWS_PER_W, do_row, 0)

    return emb_kernel(input_ids, token_type_ids, word_emb, pos_emb, type_emb,
                      ln_gamma, ln_beta)
